# baseline (speedup 1.0000x reference)
import functools

import jax
import jax.numpy as jnp
from jax import lax
from jax.experimental import pallas as pl
from jax.experimental.pallas import tpu as pltpu
from jax.experimental.pallas import tpu_sc as plsc

NUM_SEG = 1024
N_ROWS = 320000
D = 128
NC = 2
NS = 16
NW = NC * NS

R_SC = 161280
CHUNK = R_SC // NW
BLK = 120
NRING = 2
NBLK = CHUNK // BLK
PAD = 16
NVEC = CHUNK // 16
STRIPE = NUM_SEG // NS

BLK_TC = 1280
OFF_TC = R_SC // BLK_TC
NB_TC = (N_ROWS - R_SC) // BLK_TC
W_TC = 256


def _sc_body(x_hbm, ids_hbm, out_acc, out_cnt,
             ids_v, buf0, buf1, bpos, flushb, cflush, idx1, zbuf,
             sacc, scnt, sem_i, sem0, sem1):
    cid = lax.axis_index("c")
    sid = lax.axis_index("s")
    w = cid * NS + sid
    z16 = jnp.zeros((16,), jnp.float32)
    row0 = w * CHUNK

    def _blk_src(i):
        return x_hbm.at[pl.ds(row0 + i * BLK, BLK)]

    bufs = (buf0, buf1)
    sems = (sem0, sem1)

    ids_cp = pltpu.async_copy(ids_hbm.at[pl.ds(row0, CHUNK)],
                              ids_v.at[pl.ds(PAD, CHUNK)], sem_i)
    for p in range(NRING - 1):
        pltpu.async_copy(_blk_src(p), bufs[p], sems[p])

    def _z(r, carry):
        for j in range(D // 16):
            zbuf[r, pl.ds(j * 16, 16)] = z16
        return carry

    lax.fori_loop(0, STRIPE, _z, 0)
    pltpu.sync_copy(zbuf, sacc.at[pl.ds(sid * STRIPE, STRIPE)])
    pltpu.sync_copy(zbuf, scnt.at[pl.ds(sid * STRIPE, STRIPE)])
    plsc.subcore_barrier()

    ids_cp.wait()
    iota16 = lax.iota(jnp.int32, 16)
    lane0 = iota16 == 0
    first = ids_v[pl.ds(PAD, 16)][0]
    plsc.store_scatter(ids_v, [jnp.broadcast_to(jnp.int32(PAD - 1), (16,))],
                       jnp.broadcast_to(first - 1, (16,)), mask=lane0)

    def _scan(i, off):
        base = i * 16
        c = ids_v[pl.ds(base + PAD, 16)]
        p = ids_v[pl.ds(base + PAD - 1, 16)]
        m = c != p
        m32 = m.astype(jnp.int32)
        excl = plsc.cumsum(m32) - m32
        plsc.store_scatter(bpos, [off + excl], base + iota16, mask=m)
        return off + jnp.sum(m32)

    nb = lax.fori_loop(0, NVEC, _scan, jnp.int32(0))
    plsc.store_scatter(bpos, [jnp.broadcast_to(nb, (16,))],
                       jnp.broadcast_to(jnp.int32(CHUNK), (16,)), mask=lane0)

    def _flush(rs, re, acc):
        for j in range(D // 16):
            flushb[0, pl.ds(j * 16, 16)] = acc[j]
        cnt = jnp.broadcast_to((re - rs).astype(jnp.float32), (16,))
        for j in range(D // 16):
            cflush[0, pl.ds(j * 16, 16)] = cnt
        seg = ids_v[pl.ds(rs + PAD, 16)][0]
        plsc.store_scatter(idx1, [jnp.zeros((16,), jnp.int32)],
                           jnp.broadcast_to(seg, (16,)), mask=lane0)
        pltpu.sync_copy(flushb, sacc.at[idx1], add=True)
        pltpu.sync_copy(cflush, scnt.at[idx1], add=True)

    def _process(i, buf, st):
        lo = i * BLK
        hi = lo + BLK

        def _cond(s):
            return s[1] < hi

        def _piece(s):
            k, pos = s[0], s[1]
            acc = s[2:]
            bv = bpos[pl.ds(k, 16)]
            rs, re = bv[0], bv[1]
            pe = jnp.minimum(re, hi)

            @plsc.parallel_loop(pos, pe, carry=acc, unroll=4)
            def acc(r, a):
                return tuple(a[j] + buf[r - lo, pl.ds(j * 16, 16)]
                             for j in range(D // 16))

            run_done = pe == re

            def _tb(a):
                _flush(rs, re, a)
                return tuple(z16 for _ in range(D // 16))

            acc = lax.cond(run_done, _tb, lambda a: a, acc)
            k = jnp.where(run_done, k + 1, k)
            return (k, pe) + acc

        return lax.while_loop(_cond, _piece, st)

    def _phase(i, p, st):
        pltpu.make_async_copy(_blk_src(i), bufs[p], sems[p]).wait()
        pn = (p + NRING - 1) % NRING

        @pl.when(i + NRING - 1 < NBLK)
        def _start_next():
            pltpu.async_copy(_blk_src(i + NRING - 1), bufs[pn], sems[pn])

        return _process(i, bufs[p], st)

    def _round(g, st):
        for p in range(NRING):
            st = _phase(g * NRING + p, p, st)
        return st

    st0 = (jnp.int32(0), jnp.int32(0)) + tuple(z16 for _ in range(D // 16))
    lax.fori_loop(0, NBLK // NRING, _round, st0)
    plsc.subcore_barrier()

    pltpu.sync_copy(sacc.at[pl.ds(sid * STRIPE, STRIPE)], zbuf)
    pltpu.sync_copy(zbuf, out_acc.at[cid, pl.ds(sid * STRIPE, STRIPE)])
    pltpu.sync_copy(scnt.at[pl.ds(sid * STRIPE, STRIPE)], zbuf)
    pltpu.sync_copy(zbuf, out_cnt.at[cid, pl.ds(sid * STRIPE, STRIPE)])


_sc_pool = pl.kernel(
    _sc_body,
    out_type=(
        jax.ShapeDtypeStruct((NC, NUM_SEG, D), jnp.float32),
        jax.ShapeDtypeStruct((NC, NUM_SEG, D), jnp.float32),
    ),
    mesh=plsc.VectorSubcoreMesh(core_axis_name="c", subcore_axis_name="s"),
    compiler_params=pltpu.CompilerParams(needs_layout_passes=False),
    scratch_types=[
        pltpu.VMEM((CHUNK + PAD + 16,), jnp.int32),
        pltpu.VMEM((BLK, D), jnp.float32),
        pltpu.VMEM((BLK, D), jnp.float32),
        pltpu.VMEM((NUM_SEG + 48,), jnp.int32),
        pltpu.VMEM((1, D), jnp.float32),
        pltpu.VMEM((1, D), jnp.float32),
        pltpu.VMEM((1,), jnp.int32),
        pltpu.VMEM((STRIPE, D), jnp.float32),
        pltpu.VMEM_SHARED((NUM_SEG, D), jnp.float32),
        pltpu.VMEM_SHARED((NUM_SEG, D), jnp.float32),
        pltpu.SemaphoreType.DMA,
        pltpu.SemaphoreType.DMA,
        pltpu.SemaphoreType.DMA,
    ],
)


def _tc_body(ids_ref, x_ref, oa_ref, oc_ref, acc_ref, cnt_ref):
    i = pl.program_id(0)

    @pl.when(i == 0)
    def _init():
        acc_ref[...] = jnp.zeros_like(acc_ref)
        cnt_ref[...] = jnp.zeros_like(cnt_ref)

    ids = ids_ref[0, 0, :]
    x = x_ref[...]
    base = jnp.minimum((jnp.min(ids) // 8) * 8, NUM_SEG - W_TC)
    windowed = (jnp.max(ids) - base) < W_TC

    @pl.when(windowed)
    def _narrow():
        lids = ids - base
        seg = jax.lax.broadcasted_iota(jnp.int32, (W_TC, BLK_TC), 0)
        onehot = (seg == lids[None, :]).astype(jnp.float32)
        acc_ref[pl.ds(base, W_TC), :] += jax.lax.dot(
            onehot, x, preferred_element_type=jnp.float32
        )
        cnt_ref[pl.ds(base, W_TC), :] += jnp.sum(onehot, axis=1, keepdims=True)

    @pl.when(jnp.logical_not(windowed))
    def _full():
        seg = jax.lax.broadcasted_iota(jnp.int32, (NUM_SEG, BLK_TC), 0)
        onehot = (seg == ids[None, :]).astype(jnp.float32)
        acc_ref[...] += jax.lax.dot(
            onehot, x, preferred_element_type=jnp.float32
        )
        cnt_ref[...] += jnp.sum(onehot, axis=1, keepdims=True)

    @pl.when(i == NB_TC - 1)
    def _fin():
        oa_ref[...] = acc_ref[...]
        oc_ref[...] = jnp.broadcast_to(cnt_ref[...], (NUM_SEG, D))


_tc_partial = pl.pallas_call(
    _tc_body,
    grid=(NB_TC,),
    in_specs=[
        pl.BlockSpec((1, 1, BLK_TC), lambda i: (OFF_TC + i, 0, 0)),
        pl.BlockSpec((BLK_TC, D), lambda i: (OFF_TC + i, 0)),
    ],
    out_specs=[
        pl.BlockSpec((NUM_SEG, D), lambda i: (0, 0)),
        pl.BlockSpec((NUM_SEG, D), lambda i: (0, 0)),
    ],
    out_shape=[
        jax.ShapeDtypeStruct((NUM_SEG, D), jnp.float32),
        jax.ShapeDtypeStruct((NUM_SEG, D), jnp.float32),
    ],
    scratch_shapes=[
        pltpu.VMEM((NUM_SEG, D), jnp.float32),
        pltpu.VMEM((NUM_SEG, 1), jnp.float32),
    ],
)


def _combine_body(a_ref, c_ref, at_ref, ct_ref, o_ref):
    a = a_ref[0] + a_ref[1] + at_ref[...]
    c = c_ref[0] + c_ref[1] + ct_ref[...]
    o_ref[...] = a / jnp.maximum(c, 1.0)


@jax.jit
def _pool(X, ids):
    acc_sc, cnt_sc = _sc_pool(X, ids)
    ids3 = ids.reshape(N_ROWS // BLK_TC, 1, BLK_TC)
    acc_tc, cnt_tc = _tc_partial(ids3, X)
    return pl.pallas_call(
        _combine_body,
        out_shape=jax.ShapeDtypeStruct((NUM_SEG, D), jnp.float32),
    )(acc_sc, cnt_sc, acc_tc, cnt_tc)


def kernel(filtre, X, node_indicator):
    return (filtre, _pool(X, node_indicator.astype(jnp.int32)))


# speedup vs baseline: 1.1857x; 1.1857x over previous
import functools

import jax
import jax.numpy as jnp
from jax import lax
from jax.experimental import pallas as pl
from jax.experimental.pallas import tpu as pltpu
from jax.experimental.pallas import tpu_sc as plsc

NUM_SEG = 1024
N_ROWS = 320000
D = 128
NC = 2
NS = 16
NW = NC * NS

R_SC = 238080
CHUNK = R_SC // NW
BLK = 120
NRING = 2
NBLK = CHUNK // BLK
PAD = 16
NVEC = CHUNK // 16
STRIPE = NUM_SEG // NS

BLK_TC = 1280
OFF_TC = R_SC // BLK_TC
NB_TC = (N_ROWS - R_SC) // BLK_TC


def _sc_body(x_hbm, ids_hbm, out_acc, out_cnt,
             ids_v, buf0, buf1, bpos, flushb, cflush, idx1, zbuf,
             sacc, scnt, sem_i, sem0, sem1):
    cid = lax.axis_index("c")
    sid = lax.axis_index("s")
    w = cid * NS + sid
    z16 = jnp.zeros((16,), jnp.float32)
    row0 = w * CHUNK

    def _blk_src(i):
        return x_hbm.at[pl.ds(row0 + i * BLK, BLK)]

    bufs = (buf0, buf1)
    sems = (sem0, sem1)

    ids_cp = pltpu.async_copy(ids_hbm.at[pl.ds(row0, CHUNK)],
                              ids_v.at[pl.ds(PAD, CHUNK)], sem_i)
    for p in range(NRING - 1):
        pltpu.async_copy(_blk_src(p), bufs[p], sems[p])

    def _z(r, carry):
        for j in range(D // 16):
            zbuf[r, pl.ds(j * 16, 16)] = z16
        return carry

    lax.fori_loop(0, STRIPE, _z, 0)
    pltpu.sync_copy(zbuf, sacc.at[pl.ds(sid * STRIPE, STRIPE)])
    pltpu.sync_copy(zbuf, scnt.at[pl.ds(sid * STRIPE, STRIPE)])
    plsc.subcore_barrier()

    ids_cp.wait()
    iota16 = lax.iota(jnp.int32, 16)
    lane0 = iota16 == 0
    first = ids_v[pl.ds(PAD, 16)][0]
    plsc.store_scatter(ids_v, [jnp.broadcast_to(jnp.int32(PAD - 1), (16,))],
                       jnp.broadcast_to(first - 1, (16,)), mask=lane0)

    def _scan(i, off):
        base = i * 16
        c = ids_v[pl.ds(base + PAD, 16)]
        p = ids_v[pl.ds(base + PAD - 1, 16)]
        m = c != p
        m32 = m.astype(jnp.int32)
        excl = plsc.cumsum(m32) - m32
        plsc.store_scatter(bpos, [off + excl], base + iota16, mask=m)
        return off + jnp.sum(m32)

    nb = lax.fori_loop(0, NVEC, _scan, jnp.int32(0))
    plsc.store_scatter(bpos, [jnp.broadcast_to(nb, (16,))],
                       jnp.broadcast_to(jnp.int32(CHUNK), (16,)), mask=lane0)

    def _flush(rs, re, acc):
        for j in range(D // 16):
            flushb[0, pl.ds(j * 16, 16)] = acc[j]
        cnt = jnp.broadcast_to((re - rs).astype(jnp.float32), (16,))
        for j in range(D // 16):
            cflush[0, pl.ds(j * 16, 16)] = cnt
        seg = ids_v[pl.ds(rs + PAD, 16)][0]
        plsc.store_scatter(idx1, [jnp.zeros((16,), jnp.int32)],
                           jnp.broadcast_to(seg, (16,)), mask=lane0)
        pltpu.sync_copy(flushb, sacc.at[idx1], add=True)
        pltpu.sync_copy(cflush, scnt.at[idx1], add=True)

    def _process(i, buf, st):
        lo = i * BLK
        hi = lo + BLK

        def _cond(s):
            return s[1] < hi

        def _piece(s):
            k, pos = s[0], s[1]
            acc = s[2:]
            bv = bpos[pl.ds(k, 16)]
            rs, re = bv[0], bv[1]
            pe = jnp.minimum(re, hi)

            @plsc.parallel_loop(pos, pe, carry=acc, unroll=4)
            def acc(r, a):
                return tuple(a[j] + buf[r - lo, pl.ds(j * 16, 16)]
                             for j in range(D // 16))

            run_done = pe == re

            def _tb(a):
                _flush(rs, re, a)
                return tuple(z16 for _ in range(D // 16))

            acc = lax.cond(run_done, _tb, lambda a: a, acc)
            k = jnp.where(run_done, k + 1, k)
            return (k, pe) + acc

        return lax.while_loop(_cond, _piece, st)

    def _phase(i, p, st):
        pltpu.make_async_copy(_blk_src(i), bufs[p], sems[p]).wait()
        pn = (p + NRING - 1) % NRING

        @pl.when(i + NRING - 1 < NBLK)
        def _start_next():
            pltpu.async_copy(_blk_src(i + NRING - 1), bufs[pn], sems[pn])

        return _process(i, bufs[p], st)

    def _round(g, st):
        for p in range(NRING):
            st = _phase(g * NRING + p, p, st)
        return st

    st0 = (jnp.int32(0), jnp.int32(0)) + tuple(z16 for _ in range(D // 16))
    lax.fori_loop(0, NBLK // NRING, _round, st0)
    plsc.subcore_barrier()

    pltpu.sync_copy(sacc.at[pl.ds(sid * STRIPE, STRIPE)], zbuf)
    pltpu.sync_copy(zbuf, out_acc.at[cid, pl.ds(sid * STRIPE, STRIPE)])
    pltpu.sync_copy(scnt.at[pl.ds(sid * STRIPE, STRIPE)], zbuf)
    pltpu.sync_copy(zbuf, out_cnt.at[cid, pl.ds(sid * STRIPE, STRIPE)])


_sc_pool = pl.kernel(
    _sc_body,
    out_type=(
        jax.ShapeDtypeStruct((NC, NUM_SEG, D), jnp.float32),
        jax.ShapeDtypeStruct((NC, NUM_SEG, D), jnp.float32),
    ),
    mesh=plsc.VectorSubcoreMesh(core_axis_name="c", subcore_axis_name="s"),
    compiler_params=pltpu.CompilerParams(needs_layout_passes=False),
    scratch_types=[
        pltpu.VMEM((CHUNK + PAD + 16,), jnp.int32),
        pltpu.VMEM((BLK, D), jnp.float32),
        pltpu.VMEM((BLK, D), jnp.float32),
        pltpu.VMEM((NUM_SEG + 48,), jnp.int32),
        pltpu.VMEM((1, D), jnp.float32),
        pltpu.VMEM((1, D), jnp.float32),
        pltpu.VMEM((1,), jnp.int32),
        pltpu.VMEM((STRIPE, D), jnp.float32),
        pltpu.VMEM_SHARED((NUM_SEG, D), jnp.float32),
        pltpu.VMEM_SHARED((NUM_SEG, D), jnp.float32),
        pltpu.SemaphoreType.DMA,
        pltpu.SemaphoreType.DMA,
        pltpu.SemaphoreType.DMA,
    ],
)


def _tc_body(ids_ref, x_ref, oa_ref, oc_ref, acc_ref, cnt_ref):
    i = pl.program_id(0)

    @pl.when(i == 0)
    def _init():
        acc_ref[...] = jnp.zeros_like(acc_ref)
        cnt_ref[...] = jnp.zeros_like(cnt_ref)

    ids = ids_ref[0, 0, :]
    seg = jax.lax.broadcasted_iota(jnp.int32, (NUM_SEG, BLK_TC), 0)
    onehot = (seg == ids[None, :]).astype(jnp.float32)
    acc_ref[...] += jax.lax.dot(
        onehot, x_ref[...], preferred_element_type=jnp.float32
    )
    cnt_ref[...] += jnp.sum(onehot, axis=1, keepdims=True)

    @pl.when(i == NB_TC - 1)
    def _fin():
        oa_ref[...] = acc_ref[...]
        oc_ref[...] = jnp.broadcast_to(cnt_ref[...], (NUM_SEG, D))


_tc_partial = pl.pallas_call(
    _tc_body,
    grid=(NB_TC,),
    in_specs=[
        pl.BlockSpec((1, 1, BLK_TC), lambda i: (OFF_TC + i, 0, 0)),
        pl.BlockSpec((BLK_TC, D), lambda i: (OFF_TC + i, 0)),
    ],
    out_specs=[
        pl.BlockSpec((NUM_SEG, D), lambda i: (0, 0)),
        pl.BlockSpec((NUM_SEG, D), lambda i: (0, 0)),
    ],
    out_shape=[
        jax.ShapeDtypeStruct((NUM_SEG, D), jnp.float32),
        jax.ShapeDtypeStruct((NUM_SEG, D), jnp.float32),
    ],
    scratch_shapes=[
        pltpu.VMEM((NUM_SEG, D), jnp.float32),
        pltpu.VMEM((NUM_SEG, 1), jnp.float32),
    ],
)


def _combine_body(a_ref, c_ref, at_ref, ct_ref, o_ref):
    a = a_ref[0] + a_ref[1] + at_ref[...]
    c = c_ref[0] + c_ref[1] + ct_ref[...]
    o_ref[...] = a / jnp.maximum(c, 1.0)


@jax.jit
def _pool(X, ids):
    acc_sc, cnt_sc = _sc_pool(X, ids)
    ids3 = ids.reshape(N_ROWS // BLK_TC, 1, BLK_TC)
    acc_tc, cnt_tc = _tc_partial(ids3, X)
    return pl.pallas_call(
        _combine_body,
        out_shape=jax.ShapeDtypeStruct((NUM_SEG, D), jnp.float32),
    )(acc_sc, cnt_sc, acc_tc, cnt_tc)


def kernel(filtre, X, node_indicator):
    return (filtre, _pool(X, node_indicator.astype(jnp.int32)))


# speedup vs baseline: 1.4041x; 1.1842x over previous
import functools

import jax
import jax.numpy as jnp
from jax import lax
from jax.experimental import pallas as pl
from jax.experimental.pallas import tpu as pltpu
from jax.experimental.pallas import tpu_sc as plsc

NUM_SEG = 1024
N_ROWS = 320000
D = 128
NC = 2
NS = 16
NW = NC * NS

R_SC = 245760
CHUNK = R_SC // NW
BLK = 240
NRING = 2
NBLK = CHUNK // BLK
PAD = 16
NVEC = CHUNK // 16
STRIPE = NUM_SEG // NS

BLK_TC = 1280
OFF_TC = R_SC // BLK_TC
NB_TC = (N_ROWS - R_SC) // BLK_TC


def _sc_body(x_hbm, ids_hbm, out_acc, out_cnt,
             ids_v, buf0, buf1, bpos, flushb, cflush, idx1, zbuf,
             sacc, scnt, sem_i, sem0, sem1):
    cid = lax.axis_index("c")
    sid = lax.axis_index("s")
    w = cid * NS + sid
    z16 = jnp.zeros((16,), jnp.float32)
    row0 = w * CHUNK

    def _blk_src(i):
        return x_hbm.at[pl.ds(row0 + i * BLK, BLK)]

    bufs = (buf0, buf1)
    sems = (sem0, sem1)

    ids_cp = pltpu.async_copy(ids_hbm.at[pl.ds(row0, CHUNK)],
                              ids_v.at[pl.ds(PAD, CHUNK)], sem_i)
    for p in range(NRING - 1):
        pltpu.async_copy(_blk_src(p), bufs[p], sems[p])

    def _z(r, carry):
        for j in range(D // 16):
            zbuf[r, pl.ds(j * 16, 16)] = z16
        return carry

    lax.fori_loop(0, STRIPE, _z, 0)
    pltpu.sync_copy(zbuf, sacc.at[pl.ds(sid * STRIPE, STRIPE)])
    pltpu.sync_copy(zbuf, scnt.at[pl.ds(sid * STRIPE, STRIPE)])
    plsc.subcore_barrier()

    ids_cp.wait()
    iota16 = lax.iota(jnp.int32, 16)
    lane0 = iota16 == 0
    first = ids_v[pl.ds(PAD, 16)][0]
    plsc.store_scatter(ids_v, [jnp.broadcast_to(jnp.int32(PAD - 1), (16,))],
                       jnp.broadcast_to(first - 1, (16,)), mask=lane0)

    def _scan(i, off):
        base = i * 16
        c = ids_v[pl.ds(base + PAD, 16)]
        p = ids_v[pl.ds(base + PAD - 1, 16)]
        m = c != p
        m32 = m.astype(jnp.int32)
        excl = plsc.cumsum(m32) - m32
        plsc.store_scatter(bpos, [off + excl], base + iota16, mask=m)
        return off + jnp.sum(m32)

    nb = lax.fori_loop(0, NVEC, _scan, jnp.int32(0))
    plsc.store_scatter(bpos, [jnp.broadcast_to(nb, (16,))],
                       jnp.broadcast_to(jnp.int32(CHUNK), (16,)), mask=lane0)

    def _flush(rs, re, acc):
        for j in range(D // 16):
            flushb[0, pl.ds(j * 16, 16)] = acc[j]
        cnt = jnp.broadcast_to((re - rs).astype(jnp.float32), (16,))
        for j in range(D // 16):
            cflush[0, pl.ds(j * 16, 16)] = cnt
        seg = ids_v[pl.ds(rs + PAD, 16)][0]
        plsc.store_scatter(idx1, [jnp.zeros((16,), jnp.int32)],
                           jnp.broadcast_to(seg, (16,)), mask=lane0)
        pltpu.sync_copy(flushb, sacc.at[idx1], add=True)
        pltpu.sync_copy(cflush, scnt.at[idx1], add=True)

    def _process(i, buf, st):
        lo = i * BLK
        hi = lo + BLK

        def _cond(s):
            return s[1] < hi

        def _piece(s):
            k, pos = s[0], s[1]
            acc = s[2:]
            bv = bpos[pl.ds(k, 16)]
            rs, re = bv[0], bv[1]
            pe = jnp.minimum(re, hi)

            @plsc.parallel_loop(pos, pe, carry=acc, unroll=4)
            def acc(r, a):
                return tuple(a[j] + buf[r - lo, pl.ds(j * 16, 16)]
                             for j in range(D // 16))

            run_done = pe == re

            def _tb(a):
                _flush(rs, re, a)
                return tuple(z16 for _ in range(D // 16))

            acc = lax.cond(run_done, _tb, lambda a: a, acc)
            k = jnp.where(run_done, k + 1, k)
            return (k, pe) + acc

        return lax.while_loop(_cond, _piece, st)

    def _phase(i, p, st):
        pltpu.make_async_copy(_blk_src(i), bufs[p], sems[p]).wait()
        pn = (p + NRING - 1) % NRING

        @pl.when(i + NRING - 1 < NBLK)
        def _start_next():
            pltpu.async_copy(_blk_src(i + NRING - 1), bufs[pn], sems[pn])

        return _process(i, bufs[p], st)

    def _round(g, st):
        for p in range(NRING):
            st = _phase(g * NRING + p, p, st)
        return st

    st0 = (jnp.int32(0), jnp.int32(0)) + tuple(z16 for _ in range(D // 16))
    lax.fori_loop(0, NBLK // NRING, _round, st0)
    plsc.subcore_barrier()

    pltpu.sync_copy(sacc.at[pl.ds(sid * STRIPE, STRIPE)], zbuf)
    pltpu.sync_copy(zbuf, out_acc.at[cid, pl.ds(sid * STRIPE, STRIPE)])
    pltpu.sync_copy(scnt.at[pl.ds(sid * STRIPE, STRIPE)], zbuf)
    pltpu.sync_copy(zbuf, out_cnt.at[cid, pl.ds(sid * STRIPE, STRIPE)])


_sc_pool = pl.kernel(
    _sc_body,
    out_type=(
        jax.ShapeDtypeStruct((NC, NUM_SEG, D), jnp.float32),
        jax.ShapeDtypeStruct((NC, NUM_SEG, D), jnp.float32),
    ),
    mesh=plsc.VectorSubcoreMesh(core_axis_name="c", subcore_axis_name="s"),
    compiler_params=pltpu.CompilerParams(needs_layout_passes=False),
    scratch_types=[
        pltpu.VMEM((CHUNK + PAD + 16,), jnp.int32),
        pltpu.VMEM((BLK, D), jnp.float32),
        pltpu.VMEM((BLK, D), jnp.float32),
        pltpu.VMEM((NUM_SEG + 48,), jnp.int32),
        pltpu.VMEM((1, D), jnp.float32),
        pltpu.VMEM((1, D), jnp.float32),
        pltpu.VMEM((1,), jnp.int32),
        pltpu.VMEM((STRIPE, D), jnp.float32),
        pltpu.VMEM_SHARED((NUM_SEG, D), jnp.float32),
        pltpu.VMEM_SHARED((NUM_SEG, D), jnp.float32),
        pltpu.SemaphoreType.DMA,
        pltpu.SemaphoreType.DMA,
        pltpu.SemaphoreType.DMA,
    ],
)


def _tc_body(ids_ref, x_ref, oa_ref, oc_ref, acc_ref, cnt_ref):
    i = pl.program_id(0)

    @pl.when(i == 0)
    def _init():
        acc_ref[...] = jnp.zeros_like(acc_ref)
        cnt_ref[...] = jnp.zeros_like(cnt_ref)

    ids = ids_ref[0, 0, :]
    seg = jax.lax.broadcasted_iota(jnp.int32, (NUM_SEG, BLK_TC), 0)
    onehot = (seg == ids[None, :]).astype(jnp.float32)
    acc_ref[...] += jax.lax.dot(
        onehot, x_ref[...], preferred_element_type=jnp.float32
    )
    cnt_ref[...] += jnp.sum(onehot, axis=1, keepdims=True)

    @pl.when(i == NB_TC - 1)
    def _fin():
        oa_ref[...] = acc_ref[...]
        oc_ref[...] = jnp.broadcast_to(cnt_ref[...], (NUM_SEG, D))


_tc_partial = pl.pallas_call(
    _tc_body,
    grid=(NB_TC,),
    in_specs=[
        pl.BlockSpec((1, 1, BLK_TC), lambda i: (OFF_TC + i, 0, 0)),
        pl.BlockSpec((BLK_TC, D), lambda i: (OFF_TC + i, 0)),
    ],
    out_specs=[
        pl.BlockSpec((NUM_SEG, D), lambda i: (0, 0)),
        pl.BlockSpec((NUM_SEG, D), lambda i: (0, 0)),
    ],
    out_shape=[
        jax.ShapeDtypeStruct((NUM_SEG, D), jnp.float32),
        jax.ShapeDtypeStruct((NUM_SEG, D), jnp.float32),
    ],
    scratch_shapes=[
        pltpu.VMEM((NUM_SEG, D), jnp.float32),
        pltpu.VMEM((NUM_SEG, 1), jnp.float32),
    ],
)


def _combine_body(a_ref, c_ref, at_ref, ct_ref, o_ref):
    a = a_ref[0] + a_ref[1] + at_ref[...]
    c = c_ref[0] + c_ref[1] + ct_ref[...]
    o_ref[...] = a / jnp.maximum(c, 1.0)


@jax.jit
def _pool(X, ids):
    acc_sc, cnt_sc = _sc_pool(X, ids)
    ids3 = ids.reshape(N_ROWS // BLK_TC, 1, BLK_TC)
    acc_tc, cnt_tc = _tc_partial(ids3, X)
    return pl.pallas_call(
        _combine_body,
        out_shape=jax.ShapeDtypeStruct((NUM_SEG, D), jnp.float32),
    )(acc_sc, cnt_sc, acc_tc, cnt_tc)


def kernel(filtre, X, node_indicator):
    return (filtre, _pool(X, node_indicator.astype(jnp.int32)))
